# baseline (speedup 1.0000x reference)
import functools

import jax
import jax.numpy as jnp
from jax import lax
from jax.experimental import pallas as pl
from jax.experimental.pallas import tpu as pltpu
from jax.experimental.pallas import tpu_sc as plsc

_N = 10000
_E = 320000
_D = 128
_SH = 16
_NRBF = 8
_HID = 64
_RC = 5.0
_AVG_DEG = 32.0

_NC = 2
_NS = 16
_NW = _NC * _NS
_C = 64
_STEPS = 160
_EP = _NW * _STEPS * _C
_G = 8
_EH = _EP // _G
_STEPS_H = _STEPS // _G
_ACC_N = _N + 16
_RPT = 640
_RPT_LAST = _ACC_N - 15 * _RPT
_ZROWS = 8

_TE = 2560
_TN = 2000


def _coeff_body(el_ref, evT_ref, ea_ref, W1_ref, b1_ref, W2_ref, b2_ref,
                Wsh_ref, Wdir_ref, out_ref):
    r = el_ref[0].reshape(1, _TE)
    u = jnp.sqrt(2.0 / _RC) / (r + 1e-6)
    n = lax.broadcasted_iota(jnp.int32, (_NRBF, 1), 0).astype(jnp.float32) + 1.0
    thetaT = n * ((jnp.pi / _RC) * r)
    rbfT = jnp.sin(thetaT) * u
    h = lax.dot_general(rbfT, W1_ref[...],
                        (((0,), (0,)), ((), ()))) + b1_ref[...]
    h = h * jax.nn.sigmoid(h)
    radial = h @ W2_ref[...] + b2_ref[...]
    evT = evT_ref[...]
    inv = 1.0 / (jnp.sqrt(jnp.sum(evT * evT, axis=0, keepdims=True)) + 1e-6)
    dirsT = evT * inv
    sh_mix = (ea_ref[...] @ Wsh_ref[...]
              + lax.dot_general(dirsT, Wdir_ref[...],
                                (((0,), (0,)), ((), ()))))
    out_ref[...] = sh_mix * radial


def _coeff(edge_length, edge_vec, edge_attr, W1, b1, W2, b2, W_sh, W_dir,
           half):
    grid = (_EH // _TE,)
    nreal = _E // _TE - 1
    clamp = lambda i: jnp.minimum(i + half * (_EH // _TE), nreal)
    full = lambda shape: pl.BlockSpec(shape, lambda i: (0, 0))
    return pl.pallas_call(
        _coeff_body,
        grid=grid,
        in_specs=[
            pl.BlockSpec((1, _TE // 128, 128), lambda i: (clamp(i), 0, 0)),
            pl.BlockSpec((3, _TE), lambda i: (0, clamp(i))),
            pl.BlockSpec((_TE, _SH), lambda i: (clamp(i), 0)),
            full((_NRBF, _HID)),
            full((1, _HID)),
            full((_HID, _D)),
            full((1, _D)),
            full((_SH, _D)),
            full((3, _D)),
        ],
        out_specs=pl.BlockSpec((_TE, _D), lambda i: (i, 0)),
        out_shape=jax.ShapeDtypeStruct((_EH, _D), jnp.float32),
    )(edge_length.reshape(_E // _TE, _TE // 128, 128), edge_vec.T, edge_attr,
      W1, b1.reshape(1, _HID), W2, b2.reshape(1, _D), W_sh, W_dir)


def _sc_body_impl(half, x_hbm, coeff_hbm, src_hbm, dst_hbm, out_hbm,
             si0, si1, si2, si3, di0, di1, di2, di3,
             hj0, hj1, cf0, cf1, ms0, ms1, acc,
             i0, i1, i2, i3, g0, g1, s0, s1):
    c = lax.axis_index("c")
    s = lax.axis_index("s")
    wid = c * _NS + s
    sis, dis = (si0, si1, si2, si3), (di0, di1, di2, di3)
    hjs, cfs, msgs = (hj0, hj1), (cf0, cf1), (ms0, ms1)
    isem, gsem, ssem = (i0, i1, i2, i3), (g0, g1), (s0, s1)

    @pl.loop(0, _C)
    def _zero(r):
        for j in range(_D // 16):
            hj0[r, pl.ds(j * 16, 16)] = jnp.zeros((16,), jnp.float32)

    for k in range(_RPT // _C):
        @pl.when((s < 15) if k >= _RPT_LAST // _C else (s < 16))
        def _z():
            pltpu.sync_copy(hj0, acc.at[pl.ds(s * _RPT + k * _C, _C)])

    @pl.when(s == 15)
    def _zero_tail():
        pltpu.sync_copy(hj0.at[pl.ds(0, 32)],
                        acc.at[pl.ds(15 * _RPT + 6 * _C, 32)])

    plsc.subcore_barrier()

    base = wid * _STEPS_H
    gbase = half * (_EH // _C) + base

    def _fetch_idx(i, q):
        off = (gbase + i) * _C
        pltpu.async_copy(src_hbm.at[pl.ds(off, _C)], sis[q].at[0], isem[q])
        pltpu.async_copy(dst_hbm.at[pl.ds(off, _C)], dis[q].at[0], isem[q])

    def _fetch(i, b, q):
        pltpu.make_async_copy(src_hbm.at[pl.ds(0, _C)], sis[q].at[0],
                              isem[q]).wait()
        pltpu.make_async_copy(src_hbm.at[pl.ds(0, _C)], dis[q].at[0],
                              isem[q]).wait()
        pltpu.async_copy(x_hbm.at[sis[q].at[0]], hjs[b], gsem[b])
        pltpu.async_copy(coeff_hbm.at[pl.ds((base + i) * _C, _C)],
                         cfs[b], gsem[b])

    def _process(i, b, q, prefetch, wait_msg=True):
        pltpu.make_async_copy(x_hbm.at[pl.ds(0, _C)], hjs[b], gsem[b]).wait()
        pltpu.make_async_copy(x_hbm.at[pl.ds(0, _C)], cfs[b], gsem[b]).wait()
        if wait_msg:
            pltpu.make_async_copy(x_hbm.at[pl.ds(0, _C)], msgs[b],
                                  ssem[b]).wait()
        if prefetch is not None:
            _fetch_idx(prefetch, (q + 2) % 4)

        @pl.loop(0, _C)
        def _mul(r):
            for j in range(_D // 16):
                sl = pl.ds(j * 16, 16)
                msgs[b][r, sl] = hjs[b][r, sl] * cfs[b][r, sl]

        pltpu.async_copy(msgs[b], acc.at[dis[q].at[0]], ssem[b], add=True)
        if prefetch is not None:
            _fetch(prefetch, b, (q + 2) % 4)

    _fetch_idx(0, 0)
    _fetch_idx(1, 1)
    _fetch(0, 0, 0)
    _fetch(1, 1, 1)
    _process(0, 0, 0, 2, wait_msg=False)
    _process(1, 1, 1, 3, wait_msg=False)

    @pl.loop(0, (_STEPS_H - 8) // 4)
    def _main(m):
        ib = 4 * m + 2
        for j in range(4):
            _process(ib + j, j % 2, (2 + j) % 4, ib + j + 2)

    for j in range(4):
        i = _STEPS_H - 6 + j
        _process(i, i % 2, i % 4, i + 2)
    _process(_STEPS_H - 2, 0, (_STEPS_H - 2) % 4, None)
    _process(_STEPS_H - 1, 1, (_STEPS_H - 1) % 4, None)

    pltpu.make_async_copy(x_hbm.at[pl.ds(0, _C)], ms0, s0).wait()
    pltpu.make_async_copy(x_hbm.at[pl.ds(0, _C)], ms1, s1).wait()

    plsc.subcore_barrier()

    @pl.when(s < 15)
    def _dump():
        pltpu.sync_copy(acc.at[pl.ds(s * _RPT, _RPT)],
                        out_hbm.at[c, pl.ds(s * _RPT, _RPT)])

    @pl.when(s == 15)
    def _dump_last():
        pltpu.sync_copy(acc.at[pl.ds(15 * _RPT, _N - 15 * _RPT)],
                        out_hbm.at[c, pl.ds(15 * _RPT, _N - 15 * _RPT)])


def _sc_body_a(*refs):
    return _sc_body_impl(0, *refs)


def _sc_body_b(*refs):
    return _sc_body_impl(1, *refs)


def _sc_body_c(*refs):
    return _sc_body_impl(2, *refs)


def _sc_body_d(*refs):
    return _sc_body_impl(3, *refs)


def _sc_body_e(*refs):
    return _sc_body_impl(4, *refs)


def _sc_body_f(*refs):
    return _sc_body_impl(5, *refs)


def _sc_body_g(*refs):
    return _sc_body_impl(6, *refs)


def _sc_body_h(*refs):
    return _sc_body_impl(7, *refs)


_SC_BODIES = (_sc_body_a, _sc_body_b, _sc_body_c, _sc_body_d,
              _sc_body_e, _sc_body_f, _sc_body_g, _sc_body_h)


def _sc_aggregate(x, coeff, src1d, dst1d, half):
    mesh = plsc.VectorSubcoreMesh(core_axis_name="c", subcore_axis_name="s")
    kern = pl.kernel(
        _SC_BODIES[half],
        out_type=jax.ShapeDtypeStruct((_NC, _N, _D), jnp.float32),
        mesh=mesh,
        scratch_types=(
            [pltpu.VMEM((1, _C), jnp.int32) for _ in range(8)]
            + [pltpu.VMEM((_C, _D), jnp.float32) for _ in range(6)]
            + [pltpu.VMEM_SHARED((_ACC_N, _D), jnp.float32)]
            + [pltpu.SemaphoreType.DMA for _ in range(8)]
        ),
    )
    return kern(x, coeff, src1d, dst1d)


def _final_body(*refs):
    p_refs, (x_ref, Wout_ref, o_ref) = refs[:_G], refs[_G:]
    agg = sum(p[0] + p[1] for p in p_refs) * (1.0 / jnp.sqrt(_AVG_DEG))
    o_ref[...] = agg @ Wout_ref[...] + x_ref[...]


def _final(ps, x, W_out):
    grid = (_N // _TN,)
    return pl.pallas_call(
        _final_body,
        grid=grid,
        in_specs=(
            [pl.BlockSpec((_NC, _TN, _D), lambda i: (0, i, 0))
             for _ in range(_G)]
            + [pl.BlockSpec((_TN, _D), lambda i: (i, 0)),
               pl.BlockSpec((_D, _D), lambda i: (0, 0))]
        ),
        out_specs=pl.BlockSpec((_TN, _D), lambda i: (i, 0)),
        out_shape=jax.ShapeDtypeStruct((_N, _D), jnp.float32),
    )(*ps, x, W_out)


def kernel(x, edge_vec, edge_attr, edge_length, edge_src, edge_dst,
           W1, b1, W2, b2, W_sh, W_dir, W_out):
    src1d = jnp.pad(edge_src.astype(jnp.int32), (0, _EP - _E))
    dst1d = jnp.pad(edge_dst.astype(jnp.int32), (0, _EP - _E),
                    constant_values=_N)
    ps = []
    for h in range(_G):
        coeff_h = _coeff(edge_length, edge_vec, edge_attr, W1, b1, W2, b2,
                         W_sh, W_dir, h)
        ps.append(_sc_aggregate(x, coeff_h, src1d, dst1d, h))
    return _final(ps, x, W_out)


# speedup vs baseline: 1.0434x; 1.0434x over previous
import functools

import jax
import jax.numpy as jnp
from jax import lax
from jax.experimental import pallas as pl
from jax.experimental.pallas import tpu as pltpu
from jax.experimental.pallas import tpu_sc as plsc

_N = 10000
_E = 320000
_D = 128
_SH = 16
_NRBF = 8
_HID = 64
_RC = 5.0
_AVG_DEG = 32.0

_NC = 2
_NS = 16
_NW = _NC * _NS
_C = 64
_STEPS = 160
_EP = _NW * _STEPS * _C
_G = 4
_EH = _EP // _G
_STEPS_H = _STEPS // _G
_ACC_N = _N + 16
_RPT = 640
_RPT_LAST = _ACC_N - 15 * _RPT
_ZROWS = 8

_TE = 2560
_TN = 2000


def _coeff_body(el_ref, evT_ref, ea_ref, W1_ref, b1_ref, W2_ref, b2_ref,
                Wsh_ref, Wdir_ref, out_ref):
    r = el_ref[0].reshape(1, _TE)
    u = jnp.sqrt(2.0 / _RC) / (r + 1e-6)
    n = lax.broadcasted_iota(jnp.int32, (_NRBF, 1), 0).astype(jnp.float32) + 1.0
    thetaT = n * ((jnp.pi / _RC) * r)
    rbfT = jnp.sin(thetaT) * u
    h = lax.dot_general(rbfT, W1_ref[...],
                        (((0,), (0,)), ((), ()))) + b1_ref[...]
    h = h * jax.nn.sigmoid(h)
    radial = h @ W2_ref[...] + b2_ref[...]
    evT = evT_ref[...]
    inv = 1.0 / (jnp.sqrt(jnp.sum(evT * evT, axis=0, keepdims=True)) + 1e-6)
    dirsT = evT * inv
    sh_mix = (ea_ref[...] @ Wsh_ref[...]
              + lax.dot_general(dirsT, Wdir_ref[...],
                                (((0,), (0,)), ((), ()))))
    out_ref[...] = sh_mix * radial


def _coeff(edge_length, edge_vec, edge_attr, W1, b1, W2, b2, W_sh, W_dir,
           half):
    grid = (_EH // _TE,)
    nreal = _E // _TE - 1
    clamp = lambda i: jnp.minimum(i + half * (_EH // _TE), nreal)
    full = lambda shape: pl.BlockSpec(shape, lambda i: (0, 0))
    return pl.pallas_call(
        _coeff_body,
        grid=grid,
        in_specs=[
            pl.BlockSpec((1, _TE // 128, 128), lambda i: (clamp(i), 0, 0)),
            pl.BlockSpec((3, _TE), lambda i: (0, clamp(i))),
            pl.BlockSpec((_TE, _SH), lambda i: (clamp(i), 0)),
            full((_NRBF, _HID)),
            full((1, _HID)),
            full((_HID, _D)),
            full((1, _D)),
            full((_SH, _D)),
            full((3, _D)),
        ],
        out_specs=pl.BlockSpec((_TE, _D), lambda i: (i, 0)),
        out_shape=jax.ShapeDtypeStruct((_EH, _D), jnp.float32),
    )(edge_length.reshape(_E // _TE, _TE // 128, 128), edge_vec.T, edge_attr,
      W1, b1.reshape(1, _HID), W2, b2.reshape(1, _D), W_sh, W_dir)


def _sc_body_impl(half, x_hbm, coeff_hbm, src_hbm, dst_hbm, out_hbm,
             si0, si1, si2, si3, di0, di1, di2, di3,
             hj0, hj1, cf0, cf1, ms0, ms1, acc,
             i0, i1, i2, i3, g0, g1, s0, s1):
    c = lax.axis_index("c")
    s = lax.axis_index("s")
    wid = c * _NS + s
    sis, dis = (si0, si1, si2, si3), (di0, di1, di2, di3)
    hjs, cfs, msgs = (hj0, hj1), (cf0, cf1), (ms0, ms1)
    isem, gsem, ssem = (i0, i1, i2, i3), (g0, g1), (s0, s1)

    @pl.loop(0, _C)
    def _zero(r):
        for j in range(_D // 16):
            hj0[r, pl.ds(j * 16, 16)] = jnp.zeros((16,), jnp.float32)

    for k in range(_RPT // _C):
        @pl.when((s < 15) if k >= _RPT_LAST // _C else (s < 16))
        def _z():
            pltpu.sync_copy(hj0, acc.at[pl.ds(s * _RPT + k * _C, _C)])

    @pl.when(s == 15)
    def _zero_tail():
        pltpu.sync_copy(hj0.at[pl.ds(0, 32)],
                        acc.at[pl.ds(15 * _RPT + 6 * _C, 32)])

    plsc.subcore_barrier()

    base = wid * _STEPS_H
    gbase = half * (_EH // _C) + base

    def _fetch_idx(i, q):
        off = (gbase + i) * _C
        pltpu.async_copy(src_hbm.at[pl.ds(off, _C)], sis[q].at[0], isem[q])
        pltpu.async_copy(dst_hbm.at[pl.ds(off, _C)], dis[q].at[0], isem[q])

    def _fetch(i, b, q):
        pltpu.make_async_copy(src_hbm.at[pl.ds(0, _C)], sis[q].at[0],
                              isem[q]).wait()
        pltpu.make_async_copy(src_hbm.at[pl.ds(0, _C)], dis[q].at[0],
                              isem[q]).wait()
        pltpu.async_copy(x_hbm.at[sis[q].at[0]], hjs[b], gsem[b])
        pltpu.async_copy(coeff_hbm.at[pl.ds((base + i) * _C, _C)],
                         cfs[b], gsem[b])

    def _process(i, b, q, prefetch, wait_msg=True):
        pltpu.make_async_copy(x_hbm.at[pl.ds(0, _C)], hjs[b], gsem[b]).wait()
        pltpu.make_async_copy(x_hbm.at[pl.ds(0, _C)], cfs[b], gsem[b]).wait()
        if wait_msg:
            pltpu.make_async_copy(x_hbm.at[pl.ds(0, _C)], msgs[b],
                                  ssem[b]).wait()
        if prefetch is not None:
            _fetch_idx(prefetch, (q + 2) % 4)

        @pl.loop(0, _C)
        def _mul(r):
            for j in range(_D // 16):
                sl = pl.ds(j * 16, 16)
                msgs[b][r, sl] = hjs[b][r, sl] * cfs[b][r, sl]

        pltpu.async_copy(msgs[b], acc.at[dis[q].at[0]], ssem[b], add=True)
        if prefetch is not None:
            _fetch(prefetch, b, (q + 2) % 4)

    _fetch_idx(0, 0)
    _fetch_idx(1, 1)
    _fetch(0, 0, 0)
    _fetch(1, 1, 1)
    _process(0, 0, 0, 2, wait_msg=False)
    _process(1, 1, 1, 3, wait_msg=False)

    @pl.loop(0, (_STEPS_H - 8) // 4)
    def _main(m):
        ib = 4 * m + 2
        for j in range(4):
            _process(ib + j, j % 2, (2 + j) % 4, ib + j + 2)

    for j in range(4):
        i = _STEPS_H - 6 + j
        _process(i, i % 2, i % 4, i + 2)
    _process(_STEPS_H - 2, 0, (_STEPS_H - 2) % 4, None)
    _process(_STEPS_H - 1, 1, (_STEPS_H - 1) % 4, None)

    pltpu.make_async_copy(x_hbm.at[pl.ds(0, _C)], ms0, s0).wait()
    pltpu.make_async_copy(x_hbm.at[pl.ds(0, _C)], ms1, s1).wait()

    plsc.subcore_barrier()

    @pl.when(s < 15)
    def _dump():
        pltpu.sync_copy(acc.at[pl.ds(s * _RPT, _RPT)],
                        out_hbm.at[c, pl.ds(s * _RPT, _RPT)])

    @pl.when(s == 15)
    def _dump_last():
        pltpu.sync_copy(acc.at[pl.ds(15 * _RPT, _N - 15 * _RPT)],
                        out_hbm.at[c, pl.ds(15 * _RPT, _N - 15 * _RPT)])


def _sc_body_a(*refs):
    return _sc_body_impl(0, *refs)


def _sc_body_b(*refs):
    return _sc_body_impl(1, *refs)


def _sc_body_c(*refs):
    return _sc_body_impl(2, *refs)


def _sc_body_d(*refs):
    return _sc_body_impl(3, *refs)


_SC_BODIES = (_sc_body_a, _sc_body_b, _sc_body_c, _sc_body_d)


def _sc_aggregate(x, coeff, src1d, dst1d, half):
    mesh = plsc.VectorSubcoreMesh(core_axis_name="c", subcore_axis_name="s")
    kern = pl.kernel(
        _SC_BODIES[half],
        out_type=jax.ShapeDtypeStruct((_NC, _N, _D), jnp.float32),
        mesh=mesh,
        scratch_types=(
            [pltpu.VMEM((1, _C), jnp.int32) for _ in range(8)]
            + [pltpu.VMEM((_C, _D), jnp.float32) for _ in range(6)]
            + [pltpu.VMEM_SHARED((_ACC_N, _D), jnp.float32)]
            + [pltpu.SemaphoreType.DMA for _ in range(8)]
        ),
    )
    return kern(x, coeff, src1d, dst1d)


def _final_body(p0_ref, p1_ref, p2_ref, p3_ref, x_ref, Wout_ref, o_ref):
    agg = ((p0_ref[0] + p0_ref[1]) + (p1_ref[0] + p1_ref[1])
           + (p2_ref[0] + p2_ref[1]) + (p3_ref[0] + p3_ref[1])) \
        * (1.0 / jnp.sqrt(_AVG_DEG))
    o_ref[...] = agg @ Wout_ref[...] + x_ref[...]


def _final(ps, x, W_out):
    grid = (_N // _TN,)
    return pl.pallas_call(
        _final_body,
        grid=grid,
        in_specs=(
            [pl.BlockSpec((_NC, _TN, _D), lambda i: (0, i, 0))
             for _ in range(_G)]
            + [pl.BlockSpec((_TN, _D), lambda i: (i, 0)),
               pl.BlockSpec((_D, _D), lambda i: (0, 0))]
        ),
        out_specs=pl.BlockSpec((_TN, _D), lambda i: (i, 0)),
        out_shape=jax.ShapeDtypeStruct((_N, _D), jnp.float32),
    )(*ps, x, W_out)


def kernel(x, edge_vec, edge_attr, edge_length, edge_src, edge_dst,
           W1, b1, W2, b2, W_sh, W_dir, W_out):
    src1d = jnp.pad(edge_src.astype(jnp.int32), (0, _EP - _E))
    dst1d = jnp.pad(edge_dst.astype(jnp.int32), (0, _EP - _E),
                    constant_values=_N)
    ps = []
    for h in range(_G):
        coeff_h = _coeff(edge_length, edge_vec, edge_attr, W1, b1, W2, b2,
                         W_sh, W_dir, h)
        ps.append(_sc_aggregate(x, coeff_h, src1d, dst1d, h))
    return _final(ps, x, W_out)
